# initial kernel scaffold (unmeasured)
import jax
import jax.numpy as jnp
from jax import lax
from jax.experimental import pallas as pl
from jax.experimental.pallas import tpu as pltpu

N_DEV = 8


def kernel(x, w_mat, scale_x, scale_w):
    m_per, k = x.shape
    n = w_mat.shape[1]
    n_per = n // N_DEV
    m = m_per * N_DEV

    s = (scale_x * scale_w).astype(jnp.float32)

    def body(s_ref, x_ref, w_ref, out_ref, xq_ref, send_buf,
             send_sems, recv_sems):
        j = pl.program_id(0)
        my = lax.axis_index("i")
        scale = s_ref[0]

        @pl.when(j == 0)
        def _():
            xq_ref[...] = x_ref[...].astype(jnp.float8_e4m3fn)

        wq = w_ref[...].astype(jnp.float8_e4m3fn)
        p = jnp.dot(xq_ref[...], wq,
                    preferred_element_type=jnp.float32) * scale

        @pl.when(j == my)
        def _():
            out_ref[pl.ds(my * m_per, m_per), :] = p

        @pl.when(j != my)
        def _():
            send_buf[pl.ds(j, 1)] = p[None, :, :]
            rdma = pltpu.make_async_remote_copy(
                src_ref=send_buf.at[j],
                dst_ref=out_ref.at[pl.ds(my * m_per, m_per), :],
                send_sem=send_sems.at[j],
                recv_sem=recv_sems.at[my],
                device_id=(j,),
                device_id_type=pl.DeviceIdType.MESH,
            )
            rdma.start()

        @pl.when(j == N_DEV - 1)
        def _():
            for t in range(N_DEV):
                @pl.when(t != my)
                def _w():
                    pltpu.make_async_remote_copy(
                        src_ref=send_buf.at[t],
                        dst_ref=out_ref.at[pl.ds(0, m_per), :],
                        send_sem=send_sems.at[t],
                        recv_sem=recv_sems.at[t],
                        device_id=(t,),
                        device_id_type=pl.DeviceIdType.MESH,
                    ).wait_send()
                    pltpu.make_async_remote_copy(
                        src_ref=send_buf.at[t],
                        dst_ref=out_ref.at[pl.ds(t * m_per, m_per), :],
                        send_sem=send_sems.at[t],
                        recv_sem=recv_sems.at[t],
                        device_id=(t,),
                        device_id_type=pl.DeviceIdType.MESH,
                    ).wait_recv()

    return pl.pallas_call(
        body,
        grid=(N_DEV,),
        in_specs=[
            pl.BlockSpec(memory_space=pltpu.SMEM),
            pl.BlockSpec((m_per, k), lambda j: (0, 0)),
            pl.BlockSpec((k, n_per), lambda j: (0, j)),
        ],
        out_specs=pl.BlockSpec((m, n_per), lambda j: (0, 0)),
        out_shape=jax.ShapeDtypeStruct((m, n_per), jnp.float32),
        scratch_shapes=[
            pltpu.VMEM((m_per, k), jnp.float8_e4m3fn),
            pltpu.VMEM((N_DEV, m_per, n_per), jnp.float32),
            pltpu.SemaphoreType.DMA((N_DEV,)),
            pltpu.SemaphoreType.DMA((N_DEV,)),
        ],
        compiler_params=pltpu.CompilerParams(
            dimension_semantics=("arbitrary",),
            collective_id=0,
        ),
    )(s, x, w_mat)


# baseline (device time: 253454 ns/iter reference)
import jax
import jax.numpy as jnp
from jax import lax
from jax.experimental import pallas as pl
from jax.experimental.pallas import tpu as pltpu

N_DEV = 8
KH = 2


def kernel(x, w_mat, scale_x, scale_w):
    m_per, k = x.shape
    n = w_mat.shape[1]
    n_per = n // N_DEV
    n_half = n_per // KH
    m = m_per * N_DEV

    s = (scale_x * scale_w).astype(jnp.float32)

    def body(s_ref, x_ref, w_ref, out_ref, xq_ref, send_buf,
             send_sems, recv_sems):
        j = pl.program_id(0)
        my = lax.axis_index("i")
        t = j // KH
        h = j % KH
        scale = s_ref[0]

        @pl.when(j == 0)
        def _():
            xq_ref[...] = x_ref[...].astype(jnp.float8_e4m3fn)

        p = jnp.dot(xq_ref[...], w_ref[...].astype(jnp.float8_e4m3fn),
                    preferred_element_type=jnp.float32) * scale

        is_mine = t == my
        o = t - jnp.where(t > my, 1, 0)
        slot = o % 2

        @pl.when(is_mine)
        def _():
            out_ref[pl.ds(my * m_per, m_per), pl.ds(h * n_half, n_half)] = p

        @pl.when(jnp.logical_not(is_mine))
        def _():
            @pl.when((h == 0) & (o >= 2))
            def _():
                pltpu.make_async_remote_copy(
                    src_ref=send_buf.at[slot],
                    dst_ref=out_ref.at[pl.ds(0, m_per), :],
                    send_sem=send_sems.at[slot],
                    recv_sem=recv_sems.at[slot],
                    device_id=(t,),
                    device_id_type=pl.DeviceIdType.MESH,
                ).wait_send()

            send_buf[pl.ds(slot, 1), :, pl.ds(h * n_half, n_half)] = (
                p[None, :, :])

            @pl.when(h == KH - 1)
            def _():
                pltpu.make_async_remote_copy(
                    src_ref=send_buf.at[slot],
                    dst_ref=out_ref.at[pl.ds(my * m_per, m_per), :],
                    send_sem=send_sems.at[slot],
                    recv_sem=recv_sems.at[my],
                    device_id=(t,),
                    device_id_type=pl.DeviceIdType.MESH,
                ).start()

        @pl.when(j == KH * N_DEV - 1)
        def _():
            for sl in range(2):
                pltpu.make_async_remote_copy(
                    src_ref=send_buf.at[sl],
                    dst_ref=out_ref.at[pl.ds(0, m_per), :],
                    send_sem=send_sems.at[sl],
                    recv_sem=recv_sems.at[sl],
                    device_id=(0,),
                    device_id_type=pl.DeviceIdType.MESH,
                ).wait_send()
            for src in range(N_DEV):
                @pl.when(src != my)
                def _w():
                    pltpu.make_async_remote_copy(
                        src_ref=send_buf.at[0],
                        dst_ref=out_ref.at[pl.ds(src * m_per, m_per), :],
                        send_sem=send_sems.at[0],
                        recv_sem=recv_sems.at[src],
                        device_id=(src,),
                        device_id_type=pl.DeviceIdType.MESH,
                    ).wait_recv()

    return pl.pallas_call(
        body,
        grid=(KH * N_DEV,),
        in_specs=[
            pl.BlockSpec(memory_space=pltpu.SMEM),
            pl.BlockSpec((m_per, k), lambda j: (0, 0)),
            pl.BlockSpec((k, n_half), lambda j: (0, j)),
        ],
        out_specs=pl.BlockSpec((m, n_per), lambda j: (0, 0)),
        out_shape=jax.ShapeDtypeStruct((m, n_per), jnp.float32),
        scratch_shapes=[
            pltpu.VMEM((m_per, k), jnp.float8_e4m3fn),
            pltpu.VMEM((2, m_per, n_per), jnp.float32),
            pltpu.SemaphoreType.DMA((2,)),
            pltpu.SemaphoreType.DMA((N_DEV,)),
        ],
        compiler_params=pltpu.CompilerParams(
            dimension_semantics=("arbitrary",),
            vmem_limit_bytes=60 * 1024 * 1024,
        ),
    )(s, x, w_mat)
